# Initial kernel scaffold; baseline (speedup 1.0000x reference)
import jax
import jax.numpy as jnp
from jax.experimental import pallas as pl


def kernel(features, labels, centers):
    raise NotImplementedError("write your pallas kernel here")


# speedup vs baseline: 1.0233x; 1.0233x over previous
import functools

import jax
import jax.numpy as jnp
from jax import lax
from jax.experimental import pallas as pl
from jax.experimental.pallas import tpu as pltpu
from jax.experimental.pallas import tpu_sc as plsc

_B = 4096
_D = 512
_L = 16
_NC = 2
_NS = 16
_NW = _NC * _NS
_ROWS = _B // _NW
_CHUNK = 64
_NCHUNK = _ROWS // _CHUNK


def _mse_body(feat_hbm, lab_hbm, cent_hbm, out_hbm, idx_v, rows_v, feat_v,
              acc_v, sem):
    wid = lax.axis_index("s") * _NC + lax.axis_index("c")
    base = wid * _ROWS
    pltpu.sync_copy(lab_hbm.at[pl.ds(base, _ROWS)], idx_v)

    acc = jnp.zeros((_L,), jnp.float32)
    for c in range(_NCHUNK):
        gather = pltpu.async_copy(
            cent_hbm.at[idx_v.at[pl.ds(c * _CHUNK, _CHUNK)]], rows_v, sem)
        pltpu.sync_copy(feat_hbm.at[pl.ds(base + c * _CHUNK, _CHUNK)], feat_v)
        gather.wait()

        def row_body(r, a):
            for col in range(0, _D, _L):
                d = feat_v[r, pl.ds(col, _L)] - rows_v[r, pl.ds(col, _L)]
                a = d * d + a
            return a

        acc = lax.fori_loop(0, _CHUNK, row_body, acc)

    acc_v[...] = acc * (1.0 / (_B * _D))
    pltpu.sync_copy(acc_v, out_hbm.at[wid])


@functools.partial(
    pl.kernel,
    out_type=jax.ShapeDtypeStruct((_NW, _L), jnp.float32),
    mesh=plsc.VectorSubcoreMesh(core_axis_name="c", subcore_axis_name="s"),
    scratch_types=[
        pltpu.VMEM((_ROWS,), jnp.int32),
        pltpu.VMEM((_CHUNK, _D), jnp.float32),
        pltpu.VMEM((_CHUNK, _D), jnp.float32),
        pltpu.VMEM((_L,), jnp.float32),
        pltpu.SemaphoreType.DMA,
    ],
)
def _mse_kernel(feat_hbm, lab_hbm, cent_hbm, out_hbm, idx_v, rows_v, feat_v,
                acc_v, sem):
    _mse_body(feat_hbm, lab_hbm, cent_hbm, out_hbm, idx_v, rows_v, feat_v,
              acc_v, sem)


def kernel(features, labels, centers):
    partials = _mse_kernel(features, labels.astype(jnp.int32), centers)
    return jnp.sum(partials)


# speedup vs baseline: 1.0784x; 1.0539x over previous
import functools

import jax
import jax.numpy as jnp
from jax import lax
from jax.experimental import pallas as pl
from jax.experimental.pallas import tpu as pltpu
from jax.experimental.pallas import tpu_sc as plsc

_B = 4096
_D = 512
_L = 16
_NC = 2
_NS = 16
_NW = _NC * _NS
_ROWS = _B // _NW
_CHUNK = 32
_NCHUNK = _ROWS // _CHUNK
_NBUF = 2


def _mse_body(feat_hbm, lab_hbm, cent_hbm, out_hbm, idx_v, rows_v, feat_v,
              acc_v, *sems):
    wid = lax.axis_index("s") * _NC + lax.axis_index("c")
    base = wid * _ROWS
    pltpu.sync_copy(lab_hbm.at[pl.ds(base, _ROWS)], idx_v)

    def start(c):
        b = c % _NBUF
        g = pltpu.async_copy(
            cent_hbm.at[idx_v.at[pl.ds(c * _CHUNK, _CHUNK)]], rows_v.at[b],
            sems[2 * b])
        f = pltpu.async_copy(
            feat_hbm.at[pl.ds(base + c * _CHUNK, _CHUNK)], feat_v.at[b],
            sems[2 * b + 1])
        return g, f

    inflight = start(0)
    acc = jnp.zeros((_L,), jnp.float32)
    for c in range(_NCHUNK):
        nxt = start(c + 1) if c + 1 < _NCHUNK else None
        inflight[0].wait()
        inflight[1].wait()
        b = c % _NBUF

        def row_body(r, a, _b=b):
            for col in range(0, _D, _L):
                d = (feat_v.at[_b])[r, pl.ds(col, _L)] - \
                    (rows_v.at[_b])[r, pl.ds(col, _L)]
                a = d * d + a
            return a

        acc = lax.fori_loop(0, _CHUNK, row_body, acc)
        inflight = nxt

    acc_v[...] = acc * (1.0 / (_B * _D))
    pltpu.sync_copy(acc_v, out_hbm.at[wid])


@functools.partial(
    pl.kernel,
    out_type=jax.ShapeDtypeStruct((_NW, _L), jnp.float32),
    mesh=plsc.VectorSubcoreMesh(core_axis_name="c", subcore_axis_name="s"),
    scratch_types=[
        pltpu.VMEM((_ROWS,), jnp.int32),
        pltpu.VMEM((_NBUF, _CHUNK, _D), jnp.float32),
        pltpu.VMEM((_NBUF, _CHUNK, _D), jnp.float32),
        pltpu.VMEM((_L,), jnp.float32),
        pltpu.SemaphoreType.DMA,
        pltpu.SemaphoreType.DMA,
        pltpu.SemaphoreType.DMA,
        pltpu.SemaphoreType.DMA,
    ],
)
def _mse_kernel(feat_hbm, lab_hbm, cent_hbm, out_hbm, idx_v, rows_v, feat_v,
                acc_v, *sems):
    _mse_body(feat_hbm, lab_hbm, cent_hbm, out_hbm, idx_v, rows_v, feat_v,
              acc_v, *sems)


def kernel(features, labels, centers):
    partials = _mse_kernel(features, labels.astype(jnp.int32), centers)
    return jnp.sum(partials)
